# baseline (speedup 1.0000x reference)
import functools

import jax
import jax.numpy as jnp
from jax import lax
from jax.experimental import pallas as pl
from jax.experimental.pallas import tpu as pltpu
from jax.experimental.pallas import tpu_sc as plsc

DIM = 16
LANE = 128


def _glove_body(i_hbm, j_hbm, wt_hbm, wtt_hbm, bt_hbm, btt_hbm, out_hbm,
                iv, jv, wblk, wtblk, bblk, btblk, outv, sem):
    sid = lax.axis_index("s")

    @pl.when(sid == 0)
    def _():
        ci = pltpu.async_copy(i_hbm, iv, sem)
        cj = pltpu.async_copy(j_hbm, jv, sem)
        ci.wait()
        cj.wait()
        zeros = jnp.zeros((DIM,), jnp.int32)
        ivec = plsc.load_gather(iv, [zeros])
        jvec = plsc.load_gather(jv, [zeros])
        si = ivec[0]
        sj = jvec[0]
        bi = pl.multiple_of((si // LANE) * LANE, LANE)
        bj = pl.multiple_of((sj // LANE) * LANE, LANE)
        c1 = pltpu.async_copy(wt_hbm.at[:, pl.ds(bi, LANE)], wblk, sem)
        c2 = pltpu.async_copy(wtt_hbm.at[:, pl.ds(bj, LANE)], wtblk, sem)
        c3 = pltpu.async_copy(bt_hbm.at[:, pl.ds(bi, LANE)], bblk, sem)
        c4 = pltpu.async_copy(btt_hbm.at[:, pl.ds(bj, LANE)], btblk, sem)
        c1.wait()
        c2.wait()
        c3.wait()
        c4.wait()
        rows = jnp.arange(DIM, dtype=jnp.int32)
        ci16 = ivec % LANE
        cj16 = jvec % LANE
        wi = plsc.load_gather(wblk, [rows, ci16])
        wj = plsc.load_gather(wtblk, [rows, cj16])
        bval = plsc.load_gather(bblk, [zeros, ci16])
        btval = plsc.load_gather(btblk, [zeros, cj16])
        dot = jnp.sum(wi * wj)
        r = dot + bval[0] + btval[0]
        outv[...] = jnp.full((DIM,), r, dtype=jnp.float32)
        pltpu.sync_copy(outv, out_hbm)


@jax.jit
def _glove_call(i1, j1, WT, WtT, bT, btT):
    mesh = plsc.VectorSubcoreMesh(
        core_axis_name="c", subcore_axis_name="s", num_cores=1)
    fn = functools.partial(
        pl.kernel,
        mesh=mesh,
        out_type=jax.ShapeDtypeStruct((DIM,), jnp.float32),
        scratch_types=[
            pltpu.VMEM((1,), jnp.int32),
            pltpu.VMEM((1,), jnp.int32),
            pltpu.VMEM((DIM, LANE), jnp.float32),
            pltpu.VMEM((DIM, LANE), jnp.float32),
            pltpu.VMEM((1, LANE), jnp.float32),
            pltpu.VMEM((1, LANE), jnp.float32),
            pltpu.VMEM((DIM,), jnp.float32),
            pltpu.SemaphoreType.DMA,
        ],
        compiler_params=pltpu.CompilerParams(
            needs_layout_passes=False, use_tc_tiling_on_sc=True),
    )(_glove_body)
    return fn(i1, j1, WT, WtT, bT, btT)


def kernel(i, j, W, W_tilde, b, b_tilde):
    i1 = jnp.reshape(i, (1,)).astype(jnp.int32)
    j1 = jnp.reshape(j, (1,)).astype(jnp.int32)
    out = _glove_call(i1, j1, W.T, W_tilde.T, b.T, b_tilde.T)
    return out[0]


# speedup vs baseline: 6.8269x; 6.8269x over previous
import functools

import jax
import jax.numpy as jnp
from jax import lax
from jax.experimental import pallas as pl
from jax.experimental.pallas import tpu as pltpu

DIM = 16
LANE = 128


def _glove_body(i_ref, j_ref, wblk, wtblk, bblk, btblk, out):
    ci = i_ref[0] % LANE
    cj = j_ref[0] % LANE
    lane2 = lax.broadcasted_iota(jnp.int32, (DIM, LANE), 1)
    lane1 = lax.broadcasted_iota(jnp.int32, (1, LANE), 1)
    mi = (lane2 == ci).astype(jnp.float32)
    mj = (lane2 == cj).astype(jnp.float32)
    wi = jnp.sum(wblk[...] * mi, axis=1)
    wj = jnp.sum(wtblk[...] * mj, axis=1)
    dot = jnp.sum(wi * wj)
    bi = jnp.sum(bblk[...] * (lane1 == ci).astype(jnp.float32))
    bj = jnp.sum(btblk[...] * (lane1 == cj).astype(jnp.float32))
    out[0, 0] = dot + bi + bj


@jax.jit
def _glove_call(i1, j1, WT, WtT, bT, btT):
    grid_spec = pltpu.PrefetchScalarGridSpec(
        num_scalar_prefetch=2,
        grid=(1,),
        in_specs=[
            pl.BlockSpec((DIM, LANE), lambda g, si, sj: (0, si[0] // LANE)),
            pl.BlockSpec((DIM, LANE), lambda g, si, sj: (0, sj[0] // LANE)),
            pl.BlockSpec((1, LANE), lambda g, si, sj: (0, si[0] // LANE)),
            pl.BlockSpec((1, LANE), lambda g, si, sj: (0, sj[0] // LANE)),
        ],
        out_specs=pl.BlockSpec(
            (1, 1), lambda g, si, sj: (0, 0), memory_space=pltpu.SMEM),
    )
    fn = pl.pallas_call(
        _glove_body,
        grid_spec=grid_spec,
        out_shape=jax.ShapeDtypeStruct((1, 1), jnp.float32),
    )
    return fn(i1, j1, WT, WtT, bT, btT)


def kernel(i, j, W, W_tilde, b, b_tilde):
    i1 = jnp.reshape(i, (1,)).astype(jnp.int32)
    j1 = jnp.reshape(j, (1,)).astype(jnp.int32)
    out = _glove_call(i1, j1, W.T, W_tilde.T, b.T, b_tilde.T)
    return out[0, 0]
